# baseline (speedup 1.0000x reference)
import functools

import jax
import jax.numpy as jnp
from jax import lax
from jax.experimental import pallas as pl
from jax.experimental.pallas import tpu as pltpu
from jax.experimental.pallas import tpu_sc as plsc

_K = 8192
_D = 32
_N = 8192
_BLK = 256
_NBLK = _N // _BLK
_CTILE = 2048
_NCT = _K // _CTILE

_BETA = 0.25
_ALPHA = 1.0


def _dist_argmin_kernel(zf_ref, emb_ref, token_ref, hist_ref):
    zf = zf_ref[...]
    emb = emb_ref[...]
    zf_sq = jnp.sum(zf * zf, axis=1, keepdims=True)
    e_sq = jnp.sum(emb * emb, axis=1)
    zf16 = zf.astype(jnp.bfloat16).astype(jnp.float32)
    dot = lax.dot_general(zf16, emb, (((1,), (1,)), ((), ())),
                          preferred_element_type=jnp.float32)
    d = (zf_sq + e_sq[None, :]) - 2.0 * dot
    acc = jnp.full((_BLK,), jnp.inf, jnp.float32)
    win = jnp.zeros((_BLK,), jnp.int32)
    for t in range(_NCT):
        dt = d[:, t * _CTILE:(t + 1) * _CTILE]
        m = jnp.min(dt, axis=1)
        ii = lax.broadcasted_iota(jnp.int32, (_BLK, _CTILE), 1)
        idx = jnp.min(jnp.where(dt == m[:, None], ii, _CTILE), axis=1) + t * _CTILE
        upd = m < acc
        win = jnp.where(upd, idx, win)
        acc = jnp.where(upd, m, acc).astype(jnp.bfloat16).astype(jnp.float32)
    token_ref[...] = win.reshape(1, 1, _BLK)

    @pl.when(pl.program_id(0) == 0)
    def _():
        hist_ref[...] = jnp.zeros((1, _K), jnp.float32)

    kk = lax.broadcasted_iota(jnp.int32, (_BLK, _K), 1)
    hist_ref[...] += jnp.sum((win[:, None] == kk).astype(jnp.float32),
                             axis=0, keepdims=True)


def _dist_argmin(zf, emb_w):
    token3, hist = pl.pallas_call(
        _dist_argmin_kernel,
        grid=(_NBLK,),
        in_specs=[
            pl.BlockSpec((_BLK, _D), lambda i: (i, 0)),
            pl.BlockSpec((_K, _D), lambda i: (0, 0)),
        ],
        out_specs=[
            pl.BlockSpec((1, 1, _BLK), lambda i: (i, 0, 0)),
            pl.BlockSpec((1, _K), lambda i: (0, 0)),
        ],
        out_shape=[
            jax.ShapeDtypeStruct((_NBLK, 1, _BLK), jnp.int32),
            jax.ShapeDtypeStruct((1, _K), jnp.float32),
        ],
    )(zf, emb_w)
    return token3.reshape(_N), hist


_SC_INFO = plsc.get_sparse_core_info()
_NC = _SC_INFO.num_cores
_NS = _SC_INFO.num_subcores
_NW = _NC * _NS
_BPW = _N // _NW
_CHUNK = 128
_NCHUNK = _BPW // _CHUNK
_ZS = _K // _NS


_DP = 128


@functools.partial(
    pl.kernel,
    mesh=plsc.VectorSubcoreMesh(core_axis_name="c", subcore_axis_name="s"),
    out_type=jax.ShapeDtypeStruct((_NW, _NCHUNK, _CHUNK, _DP), jnp.float32),
    scratch_types=[
        pltpu.VMEM((_NCHUNK, _CHUNK), jnp.int32),
        pltpu.VMEM((_NCHUNK, _CHUNK, _DP), jnp.float32),
        pltpu.SemaphoreType.DMA,
    ],
)
def _sc_gather(emb_hbm, tok_hbm, zq_hbm, idx_v, rows_v, sem):
    cid = lax.axis_index("c")
    sid = lax.axis_index("s")
    wid = sid * _NC + cid
    pltpu.sync_copy(tok_hbm.at[wid], idx_v)
    descs = [pltpu.async_copy(emb_hbm.at[idx_v.at[c]], rows_v.at[c], sem)
             for c in range(_NCHUNK)]
    for dsc in descs:
        dsc.wait()
    pltpu.sync_copy(rows_v, zq_hbm.at[wid])


def _scalars_kernel(zf_ref, zq_ref, hist_ref, loss_ref, qerr_ref, util_ref, perp_ref):
    diff = zq_ref[...] - zf_ref[...]
    s = jnp.sum(diff * diff)
    hist = hist_ref[...].reshape(_K)
    total = jnp.sum(hist)
    avg = hist / total
    m = s / jnp.float32(_N * _D)
    loss_ref[...] = (_BETA * m + _ALPHA * m).reshape(1, 1)
    qerr_ref[...] = (s / jnp.float32(_N)).reshape(1, 1)
    util_ref[...] = (jnp.sum((hist > 0).astype(jnp.float32))
                     / jnp.float32(_K)).reshape(1, 1)
    perp_ref[...] = jnp.exp(-jnp.sum(avg * jnp.log(avg + 1e-10))).reshape(1, 1)


def _scalars(zf, zq, hist):
    outs = pl.pallas_call(
        _scalars_kernel,
        out_shape=[jax.ShapeDtypeStruct((1, 1), jnp.float32)] * 4,
    )(zf, zq, hist)
    return tuple(o.reshape(()) for o in outs)


def kernel(z, emb_w, embed_prob):
    sg = lax.stop_gradient
    zp = jnp.transpose(z, (0, 2, 3, 1))
    zf = zp.reshape(-1, _D)
    token, hist = _dist_argmin(zf, emb_w)
    emb_pad = jnp.pad(emb_w, ((0, 0), (0, _DP - _D)))
    zq4d = _sc_gather(emb_pad, token.reshape(_NW, _NCHUNK, _CHUNK))
    zq = zq4d.reshape(_N, _DP)[:, :_D]
    loss, quant_error, utilization, perplexity = _scalars(zf, zq, hist)
    zq4 = zq.reshape(zp.shape)
    z_q_st = zp + sg(zq4 - zp)
    z_q_out = jnp.transpose(z_q_st, (0, 3, 1, 2))
    return (z_q_out, loss, quant_error, utilization, perplexity)


# speedup vs baseline: 1.1312x; 1.1312x over previous
import functools

import jax
import jax.numpy as jnp
from jax import lax
from jax.experimental import pallas as pl
from jax.experimental.pallas import tpu as pltpu
from jax.experimental.pallas import tpu_sc as plsc

_K = 8192
_D = 32
_N = 8192
_BLK = 256
_NBLK = _N // _BLK
_CTILE = 2048
_NCT = _K // _CTILE

_BETA = 0.25
_ALPHA = 1.0


def _dist_argmin_kernel(zf_ref, emb_ref, token_ref, hist_ref, esq_ref, e2_ref):
    @pl.when(pl.program_id(0) == 0)
    def _():
        emb0 = emb_ref[...]
        esq_ref[...] = jnp.sum(emb0 * emb0, axis=1).reshape(1, _K)
        e2_ref[...] = emb0 + emb0

    zf = zf_ref[...]
    zf_sq = jnp.sum(zf * zf, axis=1, keepdims=True)
    e_sq = esq_ref[...]
    zf16 = zf.astype(jnp.bfloat16).astype(jnp.float32)
    dot2 = lax.dot_general(zf16, e2_ref[...], (((1,), (1,)), ((), ())),
                           preferred_element_type=jnp.float32)
    d = (zf_sq + e_sq) - dot2
    acc = jnp.full((_BLK,), jnp.inf, jnp.float32)
    win = jnp.zeros((_BLK,), jnp.int32)
    for t in range(_NCT):
        dt = d[:, t * _CTILE:(t + 1) * _CTILE]
        m = jnp.min(dt, axis=1)
        ii = lax.broadcasted_iota(jnp.int32, (_BLK, _CTILE), 1)
        idx = jnp.min(jnp.where(dt == m[:, None], ii, _CTILE), axis=1) + t * _CTILE
        upd = m < acc
        win = jnp.where(upd, idx, win)
        acc = jnp.where(upd, m, acc).astype(jnp.bfloat16).astype(jnp.float32)
    token_ref[...] = win.reshape(1, 1, _BLK)

    @pl.when(pl.program_id(0) == 0)
    def _():
        hist_ref[...] = jnp.zeros((1, _K), jnp.float32)

    kk = lax.broadcasted_iota(jnp.int32, (_BLK, _K), 1)
    hist_ref[...] += jnp.sum((win[:, None] == kk).astype(jnp.float32),
                             axis=0, keepdims=True)


def _dist_argmin(zf, emb_w):
    token3, hist = pl.pallas_call(
        _dist_argmin_kernel,
        grid=(_NBLK,),
        in_specs=[
            pl.BlockSpec((_BLK, _D), lambda i: (i, 0)),
            pl.BlockSpec((_K, _D), lambda i: (0, 0)),
        ],
        out_specs=[
            pl.BlockSpec((1, 1, _BLK), lambda i: (i, 0, 0)),
            pl.BlockSpec((1, _K), lambda i: (0, 0)),
        ],
        out_shape=[
            jax.ShapeDtypeStruct((_NBLK, 1, _BLK), jnp.int32),
            jax.ShapeDtypeStruct((1, _K), jnp.float32),
        ],
        scratch_shapes=[
            pltpu.VMEM((1, _K), jnp.float32),
            pltpu.VMEM((_K, _D), jnp.float32),
        ],
    )(zf, emb_w)
    return token3.reshape(_N), hist


_SC_INFO = plsc.get_sparse_core_info()
_NC = _SC_INFO.num_cores
_NS = _SC_INFO.num_subcores
_NW = _NC * _NS
_BPW = _N // _NW
_CHUNK = 128
_NCHUNK = _BPW // _CHUNK
_ZS = _K // _NS


_DP = 128


@functools.partial(
    pl.kernel,
    mesh=plsc.VectorSubcoreMesh(core_axis_name="c", subcore_axis_name="s"),
    out_type=jax.ShapeDtypeStruct((_NW, _NCHUNK, _CHUNK, _DP), jnp.float32),
    scratch_types=[
        pltpu.VMEM((_NCHUNK, _CHUNK), jnp.int32),
        pltpu.VMEM((_NCHUNK, _CHUNK, _DP), jnp.float32),
        pltpu.SemaphoreType.DMA,
    ],
)
def _sc_gather(emb_hbm, tok_hbm, zq_hbm, idx_v, rows_v, sem):
    cid = lax.axis_index("c")
    sid = lax.axis_index("s")
    wid = sid * _NC + cid
    pltpu.sync_copy(tok_hbm.at[wid], idx_v)
    descs = [pltpu.async_copy(emb_hbm.at[idx_v.at[c]], rows_v.at[c], sem)
             for c in range(_NCHUNK)]
    for dsc in descs:
        dsc.wait()
    pltpu.sync_copy(rows_v, zq_hbm.at[wid])


def _scalars_kernel(zf_ref, zq_ref, hist_ref, loss_ref, qerr_ref, util_ref, perp_ref):
    diff = zq_ref[...] - zf_ref[...]
    s = jnp.sum(diff * diff)
    hist = hist_ref[...].reshape(_K)
    total = jnp.sum(hist)
    avg = hist / total
    m = s / jnp.float32(_N * _D)
    loss_ref[...] = (_BETA * m + _ALPHA * m).reshape(1, 1)
    qerr_ref[...] = (s / jnp.float32(_N)).reshape(1, 1)
    util_ref[...] = (jnp.sum((hist > 0).astype(jnp.float32))
                     / jnp.float32(_K)).reshape(1, 1)
    perp_ref[...] = jnp.exp(-jnp.sum(avg * jnp.log(avg + 1e-10))).reshape(1, 1)


def _scalars(zf, zq, hist):
    outs = pl.pallas_call(
        _scalars_kernel,
        out_shape=[jax.ShapeDtypeStruct((1, 1), jnp.float32)] * 4,
    )(zf, zq, hist)
    return tuple(o.reshape(()) for o in outs)


def kernel(z, emb_w, embed_prob):
    sg = lax.stop_gradient
    zp = jnp.transpose(z, (0, 2, 3, 1))
    zf = zp.reshape(-1, _D)
    token, hist = _dist_argmin(zf, emb_w)
    emb_pad = jnp.pad(emb_w, ((0, 0), (0, _DP - _D)))
    zq4d = _sc_gather(emb_pad, token.reshape(_NW, _NCHUNK, _CHUNK))
    zq = zq4d.reshape(_N, _DP)[:, :_D]
    loss, quant_error, utilization, perplexity = _scalars(zf, zq, hist)
    zq4 = zq.reshape(zp.shape)
    z_q_st = zp + sg(zq4 - zp)
    z_q_out = jnp.transpose(z_q_st, (0, 3, 1, 2))
    return (z_q_out, loss, quant_error, utilization, perplexity)


# speedup vs baseline: 1.2577x; 1.1118x over previous
import functools

import jax
import jax.numpy as jnp
from jax import lax
from jax.experimental import pallas as pl
from jax.experimental.pallas import tpu as pltpu
from jax.experimental.pallas import tpu_sc as plsc

_K = 8192
_D = 32
_N = 8192
_BLK = 256
_NBLK = _N // _BLK
_CTILE = 2048
_NCT = _K // _CTILE

_BETA = 0.25
_ALPHA = 1.0


def _dist_argmin_kernel(zf_ref, emb_ref, token_ref, hist_ref, pad_ref,
                        esq_ref, e2_ref):
    @pl.when(pl.program_id(0) == 0)
    def _():
        emb0 = emb_ref[...]
        esq_ref[...] = jnp.sum(emb0 * emb0, axis=1).reshape(1, _K)
        e2_ref[...] = emb0 + emb0
        pad_ref[...] = jnp.concatenate(
            [emb0, jnp.zeros((_K, _DP - _D), jnp.float32)], axis=1)

    zf = zf_ref[...]
    zf_sq = jnp.sum(zf * zf, axis=1, keepdims=True)
    e_sq = esq_ref[...]
    zf16 = zf.astype(jnp.bfloat16).astype(jnp.float32)
    dot2 = lax.dot_general(zf16, e2_ref[...], (((1,), (1,)), ((), ())),
                           preferred_element_type=jnp.float32)
    d = (zf_sq + e_sq) - dot2
    acc = jnp.full((_BLK,), jnp.inf, jnp.float32)
    win = jnp.zeros((_BLK,), jnp.int32)
    for t in range(_NCT):
        dt = d[:, t * _CTILE:(t + 1) * _CTILE]
        m = jnp.min(dt, axis=1)
        idx = jnp.argmin(dt, axis=1).astype(jnp.int32) + t * _CTILE
        upd = m < acc
        win = jnp.where(upd, idx, win)
        acc = jnp.where(upd, m, acc).astype(jnp.bfloat16).astype(jnp.float32)
    token_ref[...] = win.reshape(1, 1, _BLK)

    @pl.when(pl.program_id(0) == 0)
    def _():
        hist_ref[...] = jnp.zeros((1, _K), jnp.float32)

    kk = lax.broadcasted_iota(jnp.int32, (_BLK, _K), 1)
    hist_ref[...] += jnp.sum((win[:, None] == kk).astype(jnp.float32),
                             axis=0, keepdims=True)


def _dist_argmin(zf, emb_w):
    token3, hist, emb_pad = pl.pallas_call(
        _dist_argmin_kernel,
        grid=(_NBLK,),
        in_specs=[
            pl.BlockSpec((_BLK, _D), lambda i: (i, 0)),
            pl.BlockSpec((_K, _D), lambda i: (0, 0)),
        ],
        out_specs=[
            pl.BlockSpec((1, 1, _BLK), lambda i: (i, 0, 0)),
            pl.BlockSpec((1, _K), lambda i: (0, 0)),
            pl.BlockSpec((_K, _DP), lambda i: (0, 0)),
        ],
        out_shape=[
            jax.ShapeDtypeStruct((_NBLK, 1, _BLK), jnp.int32),
            jax.ShapeDtypeStruct((1, _K), jnp.float32),
            jax.ShapeDtypeStruct((_K, _DP), jnp.float32),
        ],
        scratch_shapes=[
            pltpu.VMEM((1, _K), jnp.float32),
            pltpu.VMEM((_K, _D), jnp.float32),
        ],
    )(zf, emb_w)
    return token3.reshape(_N), hist, emb_pad


_SC_INFO = plsc.get_sparse_core_info()
_NC = _SC_INFO.num_cores
_NS = _SC_INFO.num_subcores
_NW = _NC * _NS
_BPW = _N // _NW
_CHUNK = 128
_NCHUNK = _BPW // _CHUNK
_ZS = _K // _NS


_DP = 128


@functools.partial(
    pl.kernel,
    mesh=plsc.VectorSubcoreMesh(core_axis_name="c", subcore_axis_name="s"),
    out_type=jax.ShapeDtypeStruct((_NW, _NCHUNK, _CHUNK, _DP), jnp.float32),
    scratch_types=[
        pltpu.VMEM((_NCHUNK, _CHUNK), jnp.int32),
        pltpu.VMEM((_NCHUNK, _CHUNK, _DP), jnp.float32),
        pltpu.SemaphoreType.DMA,
    ],
)
def _sc_gather(emb_hbm, tok_hbm, zq_hbm, idx_v, rows_v, sem):
    cid = lax.axis_index("c")
    sid = lax.axis_index("s")
    wid = sid * _NC + cid
    pltpu.sync_copy(tok_hbm.at[wid], idx_v)
    descs = [pltpu.async_copy(emb_hbm.at[idx_v.at[c]], rows_v.at[c], sem)
             for c in range(_NCHUNK)]
    for dsc in descs:
        dsc.wait()
    pltpu.sync_copy(rows_v, zq_hbm.at[wid])


def _scalars_kernel(zf_ref, zq_ref, hist_ref, loss_ref, qerr_ref, util_ref, perp_ref):
    diff = zq_ref[...] - zf_ref[...]
    s = jnp.sum(diff * diff)
    hist = hist_ref[...].reshape(_K)
    total = jnp.sum(hist)
    avg = hist / total
    m = s / jnp.float32(_N * _D)
    loss_ref[...] = (_BETA * m + _ALPHA * m).reshape(1, 1)
    qerr_ref[...] = (s / jnp.float32(_N)).reshape(1, 1)
    util_ref[...] = (jnp.sum((hist > 0).astype(jnp.float32))
                     / jnp.float32(_K)).reshape(1, 1)
    perp_ref[...] = jnp.exp(-jnp.sum(avg * jnp.log(avg + 1e-10))).reshape(1, 1)


def _scalars(zf, zq, hist):
    outs = pl.pallas_call(
        _scalars_kernel,
        out_shape=[jax.ShapeDtypeStruct((1, 1), jnp.float32)] * 4,
    )(zf, zq, hist)
    return tuple(o.reshape(()) for o in outs)


def kernel(z, emb_w, embed_prob):
    sg = lax.stop_gradient
    zp = jnp.transpose(z, (0, 2, 3, 1))
    zf = zp.reshape(-1, _D)
    token, hist, emb_pad = _dist_argmin(zf, emb_w)
    zq4d = _sc_gather(emb_pad, token.reshape(_NW, _NCHUNK, _CHUNK))
    zq = zq4d.reshape(_N, _DP)[:, :_D]
    loss, quant_error, utilization, perplexity = _scalars(zf, zq, hist)
    zq4 = zq.reshape(zp.shape)
    z_q_st = zp + sg(zq4 - zp)
    z_q_out = jnp.transpose(z_q_st, (0, 3, 1, 2))
    return (z_q_out, loss, quant_error, utilization, perplexity)
